# Initial kernel scaffold; baseline (speedup 1.0000x reference)
import jax
import jax.numpy as jnp
from jax.experimental import pallas as pl


def kernel(feature_1, feature_2, feature_3, feature_4, feature_5, feature_6, feature_7, feature_8, feature_9, feature_10, feature_11, feature_12, feature_13, feature_14, feature_15, feature_16, feature_17, feature_18, feature_19, feature_20, feature_21, feature_22, feature_23, feature_24, feature_25, feature_26, W_feature_1, W_feature_2, W_feature_3, W_feature_4, W_feature_5, W_feature_6, W_feature_7, W_feature_8, W_feature_9, W_feature_10, W_feature_11, W_feature_12, W_feature_13, W_feature_14, W_feature_15, W_feature_16, W_feature_17, W_feature_18, W_feature_19, W_feature_20, W_feature_21, W_feature_22, W_feature_23, W_feature_24, W_feature_25, W_feature_26):
    raise NotImplementedError("write your pallas kernel here")


# speedup vs baseline: 63.2298x; 63.2298x over previous
import functools

import jax
import jax.numpy as jnp
from jax import lax
from jax.experimental import pallas as pl
from jax.experimental.pallas import tpu as pltpu
from jax.experimental.pallas import tpu_sc as plsc

_NC, _NS, _L = 2, 16, 16
_NW = _NC * _NS
_B, _H, _D = 16384, 50, 3
_E = _B * _H
_PER_W = _E // _NW
_C = 2560
_CHUNKS = _PER_W // _C
_NF = 26
_TAB_PAD = 784


def _sc_body(*refs):
    idx_hbm = refs[:_NF]
    tab_hbm = refs[_NF]
    out_hbm = refs[_NF + 1]
    idx_v, tab_v, out_v, sem = refs[_NF + 2:]

    wid = lax.axis_index("s") * _NC + lax.axis_index("c")
    pltpu.sync_copy(tab_hbm, tab_v)
    i3 = lax.iota(jnp.int32, _L) * 3

    for g in range(_CHUNKS):
        base = wid * _PER_W + g * _C
        cps = [
            pltpu.async_copy(idx_hbm[f].at[pl.ds(base, _C)], idx_v.at[f], sem)
            for f in range(_NF)
        ]
        for cp in cps:
            cp.wait()

        def body(i, carry):
            s = pl.ds(i * _L, _L)
            a0 = a1 = a2 = None
            for f in range(_NF):
                addr = idx_v[f, s] * 3 + (f * 30)
                g0 = plsc.load_gather(tab_v, [addr])
                g1 = plsc.load_gather(tab_v, [addr + 1])
                g2 = plsc.load_gather(tab_v, [addr + 2])
                a0 = g0 if a0 is None else a0 + g0
                a1 = g1 if a1 is None else a1 + g1
                a2 = g2 if a2 is None else a2 + g2
            ob = i * (3 * _L) + i3
            plsc.store_scatter(out_v, [ob], a0)
            plsc.store_scatter(out_v, [ob + 1], a1)
            plsc.store_scatter(out_v, [ob + 2], a2)
            return carry

        lax.fori_loop(0, _C // _L, body, 0)
        pltpu.sync_copy(out_v, out_hbm.at[pl.ds(base * 3, 3 * _C)])


_sc_call = functools.partial(
    pl.kernel,
    out_type=jax.ShapeDtypeStruct((_E * _D,), jnp.float32),
    mesh=plsc.VectorSubcoreMesh(
        core_axis_name="c", subcore_axis_name="s",
        num_cores=_NC, num_subcores=_NS,
    ),
    scratch_types=[
        pltpu.VMEM((_NF, _C), jnp.int32),
        pltpu.VMEM((_TAB_PAD,), jnp.float32),
        pltpu.VMEM((_D * _C,), jnp.float32),
        pltpu.SemaphoreType.DMA,
    ],
    compiler_params=pltpu.CompilerParams(needs_layout_passes=False),
)(_sc_body)


def kernel(feature_1, feature_2, feature_3, feature_4, feature_5, feature_6, feature_7, feature_8, feature_9, feature_10, feature_11, feature_12, feature_13, feature_14, feature_15, feature_16, feature_17, feature_18, feature_19, feature_20, feature_21, feature_22, feature_23, feature_24, feature_25, feature_26, W_feature_1, W_feature_2, W_feature_3, W_feature_4, W_feature_5, W_feature_6, W_feature_7, W_feature_8, W_feature_9, W_feature_10, W_feature_11, W_feature_12, W_feature_13, W_feature_14, W_feature_15, W_feature_16, W_feature_17, W_feature_18, W_feature_19, W_feature_20, W_feature_21, W_feature_22, W_feature_23, W_feature_24, W_feature_25, W_feature_26):
    feats = [feature_1, feature_2, feature_3, feature_4, feature_5, feature_6, feature_7, feature_8, feature_9, feature_10, feature_11, feature_12, feature_13, feature_14, feature_15, feature_16, feature_17, feature_18, feature_19, feature_20, feature_21, feature_22, feature_23, feature_24, feature_25, feature_26]
    tabs = [W_feature_1, W_feature_2, W_feature_3, W_feature_4, W_feature_5, W_feature_6, W_feature_7, W_feature_8, W_feature_9, W_feature_10, W_feature_11, W_feature_12, W_feature_13, W_feature_14, W_feature_15, W_feature_16, W_feature_17, W_feature_18, W_feature_19, W_feature_20, W_feature_21, W_feature_22, W_feature_23, W_feature_24, W_feature_25, W_feature_26]
    idx_flat = [f.reshape(-1) for f in feats]
    tab = jnp.concatenate(
        [w.reshape(-1) for w in tabs]
        + [jnp.zeros((_TAB_PAD - _NF * 30,), jnp.float32)]
    )
    out = _sc_call(*idx_flat, tab)
    return out.reshape(_B, _H, _D)
